# baseline (speedup 1.0000x reference)
import jax
import jax.numpy as jnp
from jax.experimental import pallas as pl

N_Q = 8
HID = 1024


def _add_kernel(x_ref, emb_ref, o_ref):
    o_ref[...] = x_ref[...] + emb_ref[...]


def kernel(x, emb_table):
    b, q, t, h = x.shape
    xf = x.reshape(b * q, t, h)
    embf = emb_table.reshape(q, 1, h)
    out = pl.pallas_call(
        _add_kernel,
        grid=(b * q,),
        in_specs=[
            pl.BlockSpec((1, t, h), lambda i: (i, 0, 0)),
            pl.BlockSpec((1, 1, h), lambda i: (i % N_Q, 0, 0)),
        ],
        out_specs=pl.BlockSpec((1, t, h), lambda i: (i, 0, 0)),
        out_shape=jax.ShapeDtypeStruct((b * q, t, h), x.dtype),
    )(xf, embf)
    return out.reshape(b, q, t, h)
